# baseline (speedup 1.0000x reference)
import functools

import jax
import jax.numpy as jnp
from jax import lax
from jax.experimental import pallas as pl
from jax.experimental.pallas import tpu as pltpu
from jax.experimental.pallas import tpu_sc as plsc

B, T, P, N, FP, FN, E = 4, 8, 2000, 10000, 32, 16, 50000
HORIZON = 12
TFP = T * FP
TFN = T * FN
IN_DIM = TFN + TFP

NC, NS = 2, 16
NW = NC * NS
PK = 2048
D = TFP * B
NPAD = 10240
BLK = 32
NPASS = NPAD // (NW * BLK)
MROWS = 40
CHUNK = 512
EPADG = 51200
SENT = BLK * PK


def _sc_build_m(mp, bt, lov, ncv, zer):
    mesh = plsc.VectorSubcoreMesh(core_axis_name="c", subcore_axis_name="s",
                                  num_cores=NC, num_subcores=NS)

    @functools.partial(
        pl.kernel,
        out_type=jax.ShapeDtypeStruct((NPAD * PK,), jnp.float32),
        mesh=mesh,
        compiler_params=pltpu.CompilerParams(needs_layout_passes=False),
        scratch_types=[
            pltpu.VMEM((MROWS * PK,), jnp.float32),
            pltpu.VMEM((CHUNK,), jnp.int32),
            pltpu.VMEM((CHUNK,), jnp.int32),
            pltpu.VMEM((16,), jnp.int32),
            pltpu.VMEM((16,), jnp.int32),
        ],
    )
    def k(mp_h, bt_h, lov_h, ncv_h, zer_h, out_h,
          mwin, mps, bts, lrow, nrow):
        cid = lax.axis_index("c")
        sid = lax.axis_index("s")
        wid = cid * NS + sid

        ones = jnp.ones((16,), jnp.float32)

        def ppass(p, carry):
            blk = p * NW + wid
            base = blk * BLK

            pltpu.sync_copy(zer_h, mwin.at[pl.ds(0, BLK * PK)])

            pltpu.sync_copy(lov_h.at[wid, p], lrow)
            pltpu.sync_copy(ncv_h.at[wid, p], nrow)
            lo8 = jnp.max(lrow[...])
            nch = jnp.max(nrow[...])

            def chunk(c, carry2):
                off = pl.multiple_of(lo8 + c * CHUNK, 8)
                pltpu.sync_copy(mp_h.at[pl.ds(off, CHUNK)], mps)
                pltpu.sync_copy(bt_h.at[pl.ds(off, CHUNK)], bts)
                for g in range(CHUNK // 16):
                    mv = mps[pl.ds(g * 16, 16)]
                    bv = bts[pl.ds(g * 16, 16)]
                    rv = mv - base
                    oob = (rv < 0) | (rv >= BLK)
                    flat = jnp.where(oob, SENT, rv * PK + bv)
                    plsc.addupdate_scatter(mwin, [flat], ones)
                return carry2

            lax.fori_loop(0, nch, chunk, 0)

            pltpu.sync_copy(mwin.at[pl.ds(0, BLK * PK)],
                            out_h.at[pl.ds(base * PK, BLK * PK)])
            return carry

        lax.fori_loop(0, NPASS, ppass, 0)

    return k(mp, bt, lov, ncv, zer)


def _tc_mlp_body(m_ref, pbf_ref, nodes_ref, w1_ref, b1_ref,
                 w2mu_ref, b2mu_ref, w2sg_ref, b2sg_ref, mu_ref, sg_ref,
                 tab_s):
    @pl.when(pl.program_id(0) == 0)
    def _():
        tab_s[pl.ds(PK - 128, 128), :] = jnp.zeros((128, D), jnp.bfloat16)
        for b in range(B):
            for t in range(T):
                tab_s[pl.ds(0, P), pl.ds((b * T + t) * FP, FP)] = pbf_ref[b, t]

    m = m_ref[...]
    s_all = jnp.dot(m.astype(jnp.bfloat16), tab_s[...],
                    preferred_element_type=jnp.float32)
    cnt = jnp.sum(m, axis=1, keepdims=True)
    r = 1.0 / jnp.maximum(cnt, 1.0)
    w1 = w1_ref[...]
    for b in range(B):
        mean_b = s_all[:, b * TFP:(b + 1) * TFP] * r
        mlp_in = jnp.concatenate([nodes_ref[b], mean_b], axis=1)
        h = jnp.dot(mlp_in, w1, preferred_element_type=jnp.float32)
        h = jnp.maximum(h + b1_ref[...], 0.0)
        u = jnp.dot(h, w2mu_ref[...],
                    preferred_element_type=jnp.float32) + b2mu_ref[...]
        v = jnp.dot(h, w2sg_ref[...],
                    preferred_element_type=jnp.float32) + b2sg_ref[...]
        sp = jnp.maximum(v, 0.0) + jnp.log1p(jnp.exp(-jnp.abs(v)))
        mu_ref[b] = u
        sg_ref[b] = sp + 1e-6


def _tc_mlp(m2d, pbf, nodes_flat, w1, b1, w2mu, b2mu, w2sg, b2sg):
    TN = 512
    grid = (NPAD // TN,)
    return pl.pallas_call(
        _tc_mlp_body,
        grid=grid,
        scratch_shapes=[pltpu.VMEM((PK, D), jnp.bfloat16)],
        in_specs=[
            pl.BlockSpec((TN, PK), lambda i: (i, 0)),
            pl.BlockSpec((B, T, P, FP), lambda i: (0, 0, 0, 0)),
            pl.BlockSpec((B, TN, TFN), lambda i: (0, i, 0)),
            pl.BlockSpec((IN_DIM, IN_DIM), lambda i: (0, 0)),
            pl.BlockSpec((1, IN_DIM), lambda i: (0, 0)),
            pl.BlockSpec((IN_DIM, HORIZON), lambda i: (0, 0)),
            pl.BlockSpec((1, HORIZON), lambda i: (0, 0)),
            pl.BlockSpec((IN_DIM, HORIZON), lambda i: (0, 0)),
            pl.BlockSpec((1, HORIZON), lambda i: (0, 0)),
        ],
        out_specs=[
            pl.BlockSpec((B, TN, HORIZON), lambda i: (0, i, 0)),
            pl.BlockSpec((B, TN, HORIZON), lambda i: (0, i, 0)),
        ],
        out_shape=[
            jax.ShapeDtypeStruct((B, N, HORIZON), jnp.float32),
            jax.ShapeDtypeStruct((B, N, HORIZON), jnp.float32),
        ],
    )(m2d, pbf, nodes_flat, w1, b1, w2mu, b2mu, w2sg, b2sg)


def kernel(patch_x, nodes_x, subgraphs_batch, subgraphs_nodes_mapper,
           W1, b1, W2, b2):
    f32, i32 = jnp.float32, jnp.int32

    pbf = patch_x.astype(jnp.bfloat16)

    mp = jnp.concatenate([subgraphs_nodes_mapper.astype(i32),
                          jnp.full((EPADG - E,), N, i32)])
    bt = jnp.concatenate([subgraphs_batch.astype(i32),
                          jnp.zeros((EPADG - E,), i32)])

    bounds = jnp.arange(NW * NPASS + 1, dtype=i32) * BLK
    pos = jnp.sum((mp[None, :] < bounds[:, None]).astype(i32), axis=1)
    lo8 = pos[:-1] & ~7
    nch = (pos[1:] - lo8 + CHUNK - 1) // CHUNK
    lo8w = lo8.reshape(NPASS, NW).T
    nchw = nch.reshape(NPASS, NW).T
    lov = jnp.broadcast_to(lo8w[:, :, None], (NW, NPASS, 16)).astype(i32)
    ncv = jnp.broadcast_to(nchw[:, :, None], (NW, NPASS, 16)).astype(i32)

    zer = jnp.zeros((BLK * PK,), f32)

    mflat = _sc_build_m(mp, bt, lov, ncv, zer)
    m2d = mflat.reshape(NPAD, PK)

    nodes_flat = nodes_x.transpose(0, 2, 1, 3).reshape(B, N, TFN)
    w2mu, w2sg = W2[:, 0::2], W2[:, 1::2]
    b2mu, b2sg = b2[0::2].reshape(1, HORIZON), b2[1::2].reshape(1, HORIZON)

    mu_pre, sg_pre = _tc_mlp(m2d, pbf, nodes_flat, W1, b1.reshape(1, IN_DIM),
                             w2mu, b2mu, w2sg, b2sg)
    return jnp.swapaxes(mu_pre, 1, 2), jnp.swapaxes(sg_pre, 1, 2)


# speedup vs baseline: 1.0509x; 1.0509x over previous
import functools

import jax
import jax.numpy as jnp
from jax import lax
from jax.experimental import pallas as pl
from jax.experimental.pallas import tpu as pltpu
from jax.experimental.pallas import tpu_sc as plsc

B, T, P, N, FP, FN, E = 4, 8, 2000, 10000, 32, 16, 50000
HORIZON = 12
TFP = T * FP
TFN = T * FN
IN_DIM = TFN + TFP

NC, NS = 2, 16
NW = NC * NS
PK = 2048
D = TFP * B
NPAD = 10240
BLK = 32
NPASS = NPAD // (NW * BLK)
MROWS = 40
CHUNK = 512
EPADG = 51200
SENT = BLK * PK


def _sc_build_m(mp, bt, lov, ncv, zer):
    mesh = plsc.VectorSubcoreMesh(core_axis_name="c", subcore_axis_name="s",
                                  num_cores=NC, num_subcores=NS)

    @functools.partial(
        pl.kernel,
        out_type=jax.ShapeDtypeStruct((NPAD * PK,), jnp.float32),
        mesh=mesh,
        compiler_params=pltpu.CompilerParams(needs_layout_passes=False),
        scratch_types=[
            pltpu.VMEM((MROWS * PK,), jnp.float32),
            pltpu.VMEM((CHUNK,), jnp.int32),
            pltpu.VMEM((CHUNK,), jnp.int32),
            pltpu.VMEM((16,), jnp.int32),
            pltpu.VMEM((16,), jnp.int32),
        ],
    )
    def k(mp_h, bt_h, lov_h, ncv_h, zer_h, out_h,
          mwin, mps, bts, lrow, nrow):
        cid = lax.axis_index("c")
        sid = lax.axis_index("s")
        wid = cid * NS + sid

        ones = jnp.ones((16,), jnp.float32)

        def ppass(p, carry):
            blk = p * NW + wid
            base = blk * BLK

            pltpu.sync_copy(zer_h, mwin.at[pl.ds(0, BLK * PK)])

            pltpu.sync_copy(lov_h.at[wid, p], lrow)
            pltpu.sync_copy(ncv_h.at[wid, p], nrow)
            lo8 = jnp.max(lrow[...])
            nch = jnp.max(nrow[...])

            def chunk(c, carry2):
                off = pl.multiple_of(lo8 + c * CHUNK, 8)
                pltpu.sync_copy(mp_h.at[pl.ds(off, CHUNK)], mps)
                pltpu.sync_copy(bt_h.at[pl.ds(off, CHUNK)], bts)
                for g in range(CHUNK // 16):
                    mv = mps[pl.ds(g * 16, 16)]
                    bv = bts[pl.ds(g * 16, 16)]
                    rv = mv - base
                    oob = (rv < 0) | (rv >= BLK)
                    flat = jnp.where(oob, SENT, rv * PK + bv)
                    plsc.addupdate_scatter(mwin, [flat], ones)
                return carry2

            lax.fori_loop(0, nch, chunk, 0)

            pltpu.sync_copy(mwin.at[pl.ds(0, BLK * PK)],
                            out_h.at[pl.ds(base * PK, BLK * PK)])
            return carry

        lax.fori_loop(0, NPASS, ppass, 0)

    return k(mp, bt, lov, ncv, zer)


def _tc_mlp_body(m_ref, tab_ref, nodes_ref, w1_ref, b1_ref,
                 w2mu_ref, b2mu_ref, w2sg_ref, b2sg_ref, mu_ref, sg_ref):
    m = m_ref[...]
    s_all = jnp.dot(m.astype(jnp.bfloat16), tab_ref[...],
                    preferred_element_type=jnp.float32)
    cnt = jnp.sum(m, axis=1, keepdims=True)
    r = 1.0 / jnp.maximum(cnt, 1.0)
    w1 = w1_ref[...]
    mlp_in = jnp.concatenate(
        [jnp.concatenate([nodes_ref[b], s_all[:, b * TFP:(b + 1) * TFP] * r],
                         axis=1) for b in range(B)], axis=0)
    h = jnp.dot(mlp_in, w1, preferred_element_type=jnp.float32)
    h = jnp.maximum(h + b1_ref[...], 0.0)
    u = jnp.dot(h, w2mu_ref[...],
                preferred_element_type=jnp.float32) + b2mu_ref[...]
    v = jnp.dot(h, w2sg_ref[...],
                preferred_element_type=jnp.float32) + b2sg_ref[...]
    sp = jnp.maximum(v, 0.0) + jnp.log1p(jnp.exp(-jnp.abs(v))) + 1e-6
    TN = u.shape[0] // B
    for b in range(B):
        mu_ref[b] = u[b * TN:(b + 1) * TN]
        sg_ref[b] = sp[b * TN:(b + 1) * TN]


def _tc_mlp(m2d, tab, nodes_flat, w1, b1, w2mu, b2mu, w2sg, b2sg):
    TN = 512
    grid = (NPAD // TN,)
    return pl.pallas_call(
        _tc_mlp_body,
        grid=grid,
        in_specs=[
            pl.BlockSpec((TN, PK), lambda i: (i, 0)),
            pl.BlockSpec((PK, D), lambda i: (0, 0)),
            pl.BlockSpec((B, TN, TFN), lambda i: (0, i, 0)),
            pl.BlockSpec((IN_DIM, IN_DIM), lambda i: (0, 0)),
            pl.BlockSpec((1, IN_DIM), lambda i: (0, 0)),
            pl.BlockSpec((IN_DIM, HORIZON), lambda i: (0, 0)),
            pl.BlockSpec((1, HORIZON), lambda i: (0, 0)),
            pl.BlockSpec((IN_DIM, HORIZON), lambda i: (0, 0)),
            pl.BlockSpec((1, HORIZON), lambda i: (0, 0)),
        ],
        out_specs=[
            pl.BlockSpec((B, TN, HORIZON), lambda i: (0, i, 0)),
            pl.BlockSpec((B, TN, HORIZON), lambda i: (0, i, 0)),
        ],
        out_shape=[
            jax.ShapeDtypeStruct((B, N, HORIZON), jnp.float32),
            jax.ShapeDtypeStruct((B, N, HORIZON), jnp.float32),
        ],
    )(m2d, tab, nodes_flat, w1, b1, w2mu, b2mu, w2sg, b2sg)


def kernel(patch_x, nodes_x, subgraphs_batch, subgraphs_nodes_mapper,
           W1, b1, W2, b2):
    f32, i32 = jnp.float32, jnp.int32

    tab = patch_x.transpose(2, 0, 1, 3).reshape(P, D)
    tab = jnp.concatenate([tab, jnp.zeros((PK - P, D), f32)])
    tab = tab.astype(jnp.bfloat16)

    mp = jnp.concatenate([subgraphs_nodes_mapper.astype(i32),
                          jnp.full((EPADG - E,), N, i32)])
    bt = jnp.concatenate([subgraphs_batch.astype(i32),
                          jnp.zeros((EPADG - E,), i32)])

    bounds = jnp.arange(NW * NPASS + 1, dtype=i32) * BLK
    pos = jnp.sum((mp[None, :] < bounds[:, None]).astype(i32), axis=1)
    lo8 = pos[:-1] & ~7
    nch = (pos[1:] - lo8 + CHUNK - 1) // CHUNK
    lo8w = lo8.reshape(NPASS, NW).T
    nchw = nch.reshape(NPASS, NW).T
    lov = jnp.broadcast_to(lo8w[:, :, None], (NW, NPASS, 16)).astype(i32)
    ncv = jnp.broadcast_to(nchw[:, :, None], (NW, NPASS, 16)).astype(i32)

    zer = jnp.zeros((BLK * PK,), f32)

    mflat = _sc_build_m(mp, bt, lov, ncv, zer)
    m2d = mflat.reshape(NPAD, PK)

    nodes_flat = nodes_x.transpose(0, 2, 1, 3).reshape(B, N, TFN)
    w2mu, w2sg = W2[:, 0::2], W2[:, 1::2]
    b2mu, b2sg = b2[0::2].reshape(1, HORIZON), b2[1::2].reshape(1, HORIZON)

    mu_pre, sg_pre = _tc_mlp(m2d, tab, nodes_flat, W1, b1.reshape(1, IN_DIM),
                             w2mu, b2mu, w2sg, b2sg)
    return jnp.swapaxes(mu_pre, 1, 2), jnp.swapaxes(sg_pre, 1, 2)
